# baseline (speedup 1.0000x reference)
import functools

import jax
import jax.numpy as jnp
from jax import lax
from jax.experimental import pallas as pl
from jax.experimental.pallas import tpu as pltpu
from jax.experimental.pallas import tpu_sc as plsc

N = 10000
E = 320000
IN_DIM = 128
H = 64
L = 8
G = 16

D = 640
NB = 10
RB = 1000

PW = 128
NPAD = 10240
NPT = NPAD // 16
EPAD = 327680
EC = 128
EROWS = EPAD // EC
EPT = EROWS // 32
NPASS = 2
PR = EPT // NPASS


def _sc_agg(p, src_rows, dst_rows, zero_rows):
    mesh = plsc.VectorSubcoreMesh(core_axis_name="c", subcore_axis_name="s")

    @functools.partial(
        pl.kernel,
        mesh=mesh,
        out_type=jax.ShapeDtypeStruct((2, NPAD, PW), jnp.float32),
        scratch_types=[
            pltpu.VMEM((PR, EC), jnp.int32),
            pltpu.VMEM((PR, EC), jnp.int32),
            pltpu.VMEM((EC, PW), jnp.float32),
            pltpu.VMEM((EC, PW), jnp.float32),
            pltpu.VMEM_SHARED((NPAD, PW), jnp.float32),
            pltpu.SemaphoreType.DMA,
            pltpu.SemaphoreType.DMA,
        ],
    )
    def body(p_hbm, src_hbm, dst_hbm, zer_hbm, out_hbm,
             sidx, didx, rows0, rows1, acc, sem0, sem1):
        c = lax.axis_index("c")
        s = lax.axis_index("s")
        wid = c * 16 + s
        zr = s * NPT
        pltpu.sync_copy(zer_hbm.at[pl.ds(zr, NPT)], acc.at[pl.ds(zr, NPT)])
        r0 = wid * EPT
        plsc.subcore_barrier()

        def run_pass(base):
            pltpu.sync_copy(src_hbm.at[pl.ds(base, PR)], sidx)
            pltpu.sync_copy(dst_hbm.at[pl.ds(base, PR)], didx)
            pltpu.async_copy(p_hbm.at[sidx.at[0]], rows0, sem0)

            def step(jj, carry):
                j0 = 2 * jj
                j1 = j0 + 1
                pltpu.async_copy(p_hbm.at[sidx.at[j1]], rows1, sem1)
                pltpu.make_async_copy(p_hbm.at[sidx.at[j0]], rows0, sem0).wait()
                pltpu.sync_copy(rows0, acc.at[didx.at[j0]], add=True)

                @pl.when(jj < PR // 2 - 1)
                def _():
                    pltpu.async_copy(p_hbm.at[sidx.at[j0 + 2]], rows0, sem0)

                pltpu.make_async_copy(p_hbm.at[sidx.at[j1]], rows1, sem1).wait()
                pltpu.sync_copy(rows1, acc.at[didx.at[j1]], add=True)
                return carry

            lax.fori_loop(0, PR // 2, step, 0)

        run_pass(r0)
        run_pass(r0 + PR)
        plsc.subcore_barrier()
        pltpu.sync_copy(acc.at[pl.ds(zr, NPT)], out_hbm.at[c, pl.ds(zr, NPT)])

    return body(p, src_rows, dst_rows, zero_rows)


def _pre_call(x, pre_w, pre_b2, lw0, rw0):
    def body(x_ref, w_ref, b_ref, lw_ref, rw_ref, emb_ref, p_ref, r_ref):
        h0 = jnp.dot(x_ref[...], w_ref[...],
                     preferred_element_type=jnp.float32) + b_ref[...]
        emb_ref[...] = jnp.concatenate(
            [h0, jnp.zeros((RB, D - H), jnp.float32)], axis=1)
        p_ref[...] = jnp.concatenate(
            [jnp.dot(h0, lw_ref[...], preferred_element_type=jnp.float32),
             jnp.zeros((RB, PW - H), jnp.float32)], axis=1)
        r_ref[...] = jnp.dot(h0, rw_ref[...], preferred_element_type=jnp.float32)

    return pl.pallas_call(
        body,
        grid=(NB,),
        in_specs=[
            pl.BlockSpec((RB, IN_DIM), lambda j: (j, 0)),
            pl.BlockSpec((IN_DIM, H), lambda j: (0, 0)),
            pl.BlockSpec((1, H), lambda j: (0, 0)),
            pl.BlockSpec((H, H), lambda j: (0, 0)),
            pl.BlockSpec((H, H), lambda j: (0, 0)),
        ],
        out_specs=[
            pl.BlockSpec((RB, D), lambda j: (j, 0)),
            pl.BlockSpec((RB, PW), lambda j: (j, 0)),
            pl.BlockSpec((RB, H), lambda j: (j, 0)),
        ],
        out_shape=[
            jax.ShapeDtypeStruct((N, D), jnp.float32),
            jax.ShapeDtypeStruct((N, PW), jnp.float32),
            jax.ShapeDtypeStruct((N, H), jnp.float32),
        ],
    )(x, pre_w, pre_b2, lw0, rw0)


def _layer_call(i, emb, parts, rprev, bprev, lw, rw):
    lo = i * H

    def body(emb_ref, p0_ref, p1_ref, rp_ref, b_ref, lw_ref, rw_ref,
             embu_ref, p_ref, r_ref):
        hnew = jnp.maximum(
            p0_ref[0, :, :H] + p1_ref[0, :, :H] + rp_ref[...] + b_ref[...], 0.0)
        e = emb_ref[...]
        embu_ref[...] = e + jnp.concatenate(
            [jnp.zeros((RB, lo), jnp.float32), hnew,
             jnp.zeros((RB, D - lo - H), jnp.float32)], axis=1)
        p_ref[...] = jnp.concatenate(
            [jnp.dot(e, lw_ref[...], preferred_element_type=jnp.float32)
             + jnp.dot(hnew, lw_ref[lo:lo + H, :],
                       preferred_element_type=jnp.float32),
             jnp.zeros((RB, PW - H), jnp.float32)], axis=1)
        r_ref[...] = (jnp.dot(e, rw_ref[...], preferred_element_type=jnp.float32)
                      + jnp.dot(hnew, rw_ref[lo:lo + H, :],
                                preferred_element_type=jnp.float32))

    return pl.pallas_call(
        body,
        grid=(NB,),
        in_specs=[
            pl.BlockSpec((RB, D), lambda j: (j, 0)),
            pl.BlockSpec((1, RB, PW), lambda j: (0, j, 0)),
            pl.BlockSpec((1, RB, PW), lambda j: (1, j, 0)),
            pl.BlockSpec((RB, H), lambda j: (j, 0)),
            pl.BlockSpec((1, H), lambda j: (0, 0)),
            pl.BlockSpec((D, H), lambda j: (0, 0)),
            pl.BlockSpec((D, H), lambda j: (0, 0)),
        ],
        out_specs=[
            pl.BlockSpec((RB, D), lambda j: (j, 0)),
            pl.BlockSpec((RB, PW), lambda j: (j, 0)),
            pl.BlockSpec((RB, H), lambda j: (j, 0)),
        ],
        out_shape=[
            jax.ShapeDtypeStruct((N, D), jnp.float32),
            jax.ShapeDtypeStruct((N, PW), jnp.float32),
            jax.ShapeDtypeStruct((N, H), jnp.float32),
        ],
        input_output_aliases={0: 0},
    )(emb, parts, parts, rprev, bprev, lw, rw)


def _final_call(emb, parts, r7, b7, batch_r, w1p, b1, w2, b2, w3, b3, w4, b4):
    def body(emb_ref, p0_ref, p1_ref, rp_ref, b_ref, bat_ref,
             w1_ref, b1_ref, w2_ref, b2_ref, w3_ref, b3_ref, w4_ref, b4_ref,
             out_ref, acc_ref, acch_ref):
        j = pl.program_id(0)

        @pl.when(j == 0)
        def _():
            acc_ref[...] = jnp.zeros((G, D), jnp.float32)
            acch_ref[...] = jnp.zeros((G, H), jnp.float32)

        h8 = jnp.maximum(
            p0_ref[0, :, :H] + p1_ref[0, :, :H] + rp_ref[...] + b_ref[...], 0.0)
        e = emb_ref[...]
        bv = bat_ref[0, 0, :]
        mask = (bv[None, :] == lax.broadcasted_iota(jnp.int32, (G, RB), 0)
                ).astype(jnp.float32)
        acc_ref[...] += jnp.dot(mask, e, preferred_element_type=jnp.float32)
        acch_ref[...] += jnp.dot(mask, h8, preferred_element_type=jnp.float32)

        @pl.when(j == NB - 1)
        def _():
            h = (jnp.dot(acc_ref[...], w1_ref[...],
                         preferred_element_type=jnp.float32)
                 + jnp.dot(acch_ref[...], w1_ref[8 * H:9 * H, :],
                           preferred_element_type=jnp.float32)
                 + b1_ref[...])
            h = jnp.where(h >= 0, h, 0.1 * h)
            h = jnp.maximum(jnp.dot(h, w2_ref[...],
                                    preferred_element_type=jnp.float32)
                            + b2_ref[...], 0.0)
            h = jnp.maximum(jnp.dot(h, w3_ref[...],
                                    preferred_element_type=jnp.float32)
                            + b3_ref[...], 0.0)
            out_ref[...] = jnp.dot(h, w4_ref[...],
                                   preferred_element_type=jnp.float32) + b4_ref[...]

    return pl.pallas_call(
        body,
        grid=(NB,),
        in_specs=[
            pl.BlockSpec((RB, D), lambda j: (j, 0)),
            pl.BlockSpec((1, RB, PW), lambda j: (0, j, 0)),
            pl.BlockSpec((1, RB, PW), lambda j: (1, j, 0)),
            pl.BlockSpec((RB, H), lambda j: (j, 0)),
            pl.BlockSpec((1, H), lambda j: (0, 0)),
            pl.BlockSpec((1, 1, RB), lambda j: (j, 0, 0)),
            pl.BlockSpec((D, H), lambda j: (0, 0)),
            pl.BlockSpec((1, H), lambda j: (0, 0)),
            pl.BlockSpec((H, H), lambda j: (0, 0)),
            pl.BlockSpec((1, H), lambda j: (0, 0)),
            pl.BlockSpec((H, 256), lambda j: (0, 0)),
            pl.BlockSpec((1, 256), lambda j: (0, 0)),
            pl.BlockSpec((256, H), lambda j: (0, 0)),
            pl.BlockSpec((1, H), lambda j: (0, 0)),
        ],
        out_specs=pl.BlockSpec((G, H), lambda j: (0, 0)),
        out_shape=jax.ShapeDtypeStruct((G, H), jnp.float32),
        scratch_shapes=[
            pltpu.VMEM((G, D), jnp.float32),
            pltpu.VMEM((G, H), jnp.float32),
        ],
    )(emb, parts, parts, r7, b7, batch_r, w1p, b1, w2, b2, w3, b3, w4, b4)


def kernel(x, edge_index, batch, learnable_skip, pre_w, pre_b,
           ll_w_0, ll_b_0, lr_w_0, ll_w_1, ll_b_1, lr_w_1,
           ll_w_2, ll_b_2, lr_w_2, ll_w_3, ll_b_3, lr_w_3,
           ll_w_4, ll_b_4, lr_w_4, ll_w_5, ll_b_5, lr_w_5,
           ll_w_6, ll_b_6, lr_w_6, ll_w_7, ll_b_7, lr_w_7,
           post_w1, post_b1, post_w2, post_b2,
           post_w3, post_b3, post_w4, post_b4):
    ll_ws = [ll_w_0, ll_w_1, ll_w_2, ll_w_3, ll_w_4, ll_w_5, ll_w_6, ll_w_7]
    ll_bs = [ll_b_0, ll_b_1, ll_b_2, ll_b_3, ll_b_4, ll_b_5, ll_b_6, ll_b_7]
    lr_ws = [lr_w_0, lr_w_1, lr_w_2, lr_w_3, lr_w_4, lr_w_5, lr_w_6, lr_w_7]

    skip = jax.nn.sigmoid(learnable_skip)
    sll, slr = [], []
    for i in range(L):
        sc = jnp.repeat(skip[i, :i + 1], H)[:, None]
        sll.append(jnp.pad(ll_ws[i] * sc, ((0, D - (i + 1) * H), (0, 0))))
        slr.append(jnp.pad(lr_ws[i] * sc, ((0, D - (i + 1) * H), (0, 0))))

    src_s, dst_s = lax.sort_key_val(edge_index[0], edge_index[1])
    src_rows = jnp.pad(src_s, (0, EPAD - E)).reshape(EROWS, EC)
    dst_rows = jnp.pad(dst_s, (0, EPAD - E),
                       constant_values=N).reshape(EROWS, EC)
    zero_rows = jnp.zeros((NPAD, PW), jnp.float32)
    batch_r = batch.reshape(NB, 1, RB)

    emb, p, r = _pre_call(x, pre_w, pre_b.reshape(1, H),
                          sll[0][:H], slr[0][:H])
    for i in range(1, L):
        parts = _sc_agg(p, src_rows, dst_rows, zero_rows)
        emb, p, r = _layer_call(i, emb, parts, r,
                                ll_bs[i - 1].reshape(1, H), sll[i], slr[i])
    parts = _sc_agg(p, src_rows, dst_rows, zero_rows)
    return _final_call(emb, parts, r, ll_bs[L - 1].reshape(1, H), batch_r,
                       jnp.pad(post_w1, ((0, D - 9 * H), (0, 0))),
                       post_b1.reshape(1, H), post_w2, post_b2.reshape(1, H),
                       post_w3, post_b3.reshape(1, 256),
                       post_w4, post_b4.reshape(1, H))


# speedup vs baseline: 1.0972x; 1.0972x over previous
import functools

import jax
import jax.numpy as jnp
from jax import lax
from jax.experimental import pallas as pl
from jax.experimental.pallas import tpu as pltpu
from jax.experimental.pallas import tpu_sc as plsc

N = 10000
E = 320000
IN_DIM = 128
H = 64
L = 8
G = 16

D = 640
NB = 10
RB = 1000

PW = 128
NPAD = 10240
NPT = NPAD // 16
EPAD = 327680
EC = 128
EROWS = EPAD // EC
EPT = EROWS // 32
NPASS = 2
PR = EPT // NPASS


def _sc_agg(p, src_rows, dst_rows, zero_rows):
    mesh = plsc.VectorSubcoreMesh(core_axis_name="c", subcore_axis_name="s")

    @functools.partial(
        pl.kernel,
        mesh=mesh,
        out_type=jax.ShapeDtypeStruct((2, NPAD, PW), jnp.float32),
        scratch_types=[
            pltpu.VMEM((PR, EC), jnp.int32),
            pltpu.VMEM((PR, EC), jnp.int32),
            pltpu.VMEM((EC, PW), jnp.float32),
            pltpu.VMEM((EC, PW), jnp.float32),
            pltpu.VMEM_SHARED((NPAD, PW), jnp.float32),
            pltpu.SemaphoreType.DMA,
            pltpu.SemaphoreType.DMA,
        ],
    )
    def body(p_hbm, src_hbm, dst_hbm, zer_hbm, out_hbm,
             sidx, didx, rows0, rows1, acc, sem0, sem1):
        c = lax.axis_index("c")
        s = lax.axis_index("s")
        wid = c * 16 + s
        zr = s * NPT
        pltpu.sync_copy(zer_hbm.at[pl.ds(zr, NPT)], acc.at[pl.ds(zr, NPT)])
        r0 = wid * EPT
        plsc.subcore_barrier()

        def run_pass(base):
            pltpu.sync_copy(src_hbm.at[pl.ds(base, PR)], sidx)
            pltpu.sync_copy(dst_hbm.at[pl.ds(base, PR)], didx)
            pltpu.async_copy(p_hbm.at[sidx.at[0]], rows0, sem0)

            def step(jj, carry):
                j0 = 2 * jj
                j1 = j0 + 1
                pltpu.async_copy(p_hbm.at[sidx.at[j1]], rows1, sem1)
                pltpu.make_async_copy(p_hbm.at[sidx.at[j0]], rows0, sem0).wait()
                pltpu.sync_copy(rows0, acc.at[didx.at[j0]], add=True)

                @pl.when(jj < PR // 2 - 1)
                def _():
                    pltpu.async_copy(p_hbm.at[sidx.at[j0 + 2]], rows0, sem0)

                pltpu.make_async_copy(p_hbm.at[sidx.at[j1]], rows1, sem1).wait()
                pltpu.sync_copy(rows1, acc.at[didx.at[j1]], add=True)
                return carry

            lax.fori_loop(0, PR // 2, step, 0)

        run_pass(r0)
        run_pass(r0 + PR)
        plsc.subcore_barrier()
        pltpu.sync_copy(acc.at[pl.ds(zr, NPT)], out_hbm.at[c, pl.ds(zr, NPT)])

    return body(p, src_rows, dst_rows, zero_rows)


def _pre_call(x, pre_w, pre_b2, lw0, rw0):
    def body(x_ref, w_ref, b_ref, lw_ref, rw_ref, emb_ref, p_ref, r_ref):
        h0 = jnp.dot(x_ref[...], w_ref[...],
                     preferred_element_type=jnp.float32) + b_ref[...]
        emb_ref[...] = jnp.concatenate(
            [h0, jnp.zeros((RB, D - H), jnp.float32)], axis=1)
        p_ref[...] = jnp.concatenate(
            [jnp.dot(h0, lw_ref[...], preferred_element_type=jnp.float32),
             jnp.zeros((RB, PW - H), jnp.float32)], axis=1)
        r_ref[...] = jnp.dot(h0, rw_ref[...], preferred_element_type=jnp.float32)

    return pl.pallas_call(
        body,
        grid=(NB,),
        in_specs=[
            pl.BlockSpec((RB, IN_DIM), lambda j: (j, 0)),
            pl.BlockSpec((IN_DIM, H), lambda j: (0, 0)),
            pl.BlockSpec((1, H), lambda j: (0, 0)),
            pl.BlockSpec((H, H), lambda j: (0, 0)),
            pl.BlockSpec((H, H), lambda j: (0, 0)),
        ],
        out_specs=[
            pl.BlockSpec((RB, D), lambda j: (j, 0)),
            pl.BlockSpec((RB, PW), lambda j: (j, 0)),
            pl.BlockSpec((RB, H), lambda j: (j, 0)),
        ],
        out_shape=[
            jax.ShapeDtypeStruct((N, D), jnp.float32),
            jax.ShapeDtypeStruct((N, PW), jnp.float32),
            jax.ShapeDtypeStruct((N, H), jnp.float32),
        ],
    )(x, pre_w, pre_b2, lw0, rw0)


def _layer_call(i, emb, parts, rprev, bprev, lw, rw):
    lo = i * H
    wp = (-((i + 1) * H) // 128) * -128

    def body(emb_ref, p0_ref, p1_ref, rp_ref, b_ref, lw_ref,
             embu_ref, p_ref):
        hnew = jnp.maximum(
            p0_ref[0, :, :H] + p1_ref[0, :, :H] + rp_ref[...] + b_ref[...], 0.0)
        e = emb_ref[...]
        pieces = [jnp.zeros((RB, lo), jnp.float32), hnew]
        if wp - lo - H:
            pieces.append(jnp.zeros((RB, wp - lo - H), jnp.float32))
        embu_ref[...] = e + jnp.concatenate(pieces, axis=1)
        p_ref[...] = jnp.concatenate(
            [jnp.dot(e, lw_ref[...], preferred_element_type=jnp.float32)
             + jnp.dot(hnew, lw_ref[lo:lo + H, :],
                       preferred_element_type=jnp.float32),
             jnp.zeros((RB, PW - H), jnp.float32)], axis=1)

    embu, p = pl.pallas_call(
        body,
        grid=(NB,),
        in_specs=[
            pl.BlockSpec((RB, wp), lambda j: (j, 0)),
            pl.BlockSpec((1, RB, PW), lambda j: (0, j, 0)),
            pl.BlockSpec((1, RB, PW), lambda j: (1, j, 0)),
            pl.BlockSpec((RB, H), lambda j: (j, 0)),
            pl.BlockSpec((1, H), lambda j: (0, 0)),
            pl.BlockSpec((wp, H), lambda j: (0, 0)),
        ],
        out_specs=[
            pl.BlockSpec((RB, wp), lambda j: (j, 0)),
            pl.BlockSpec((RB, PW), lambda j: (j, 0)),
        ],
        out_shape=[
            jax.ShapeDtypeStruct((N, D), jnp.float32),
            jax.ShapeDtypeStruct((N, PW), jnp.float32),
        ],
        input_output_aliases={0: 0},
    )(emb, parts, parts, rprev, bprev, lw[:wp])

    def rbody(embu_ref, rw_ref, r_ref):
        r_ref[...] = jnp.dot(embu_ref[...], rw_ref[...],
                             preferred_element_type=jnp.float32)

    r = pl.pallas_call(
        rbody,
        grid=(NB,),
        in_specs=[
            pl.BlockSpec((RB, wp), lambda j: (j, 0)),
            pl.BlockSpec((wp, H), lambda j: (0, 0)),
        ],
        out_specs=pl.BlockSpec((RB, H), lambda j: (j, 0)),
        out_shape=jax.ShapeDtypeStruct((N, H), jnp.float32),
    )(embu, rw[:wp])
    return embu, p, r


def _final_call(emb, parts, r7, b7, batch_r, w1p, b1, w2, b2, w3, b3, w4, b4):
    def body(emb_ref, p0_ref, p1_ref, rp_ref, b_ref, bat_ref,
             w1_ref, b1_ref, w2_ref, b2_ref, w3_ref, b3_ref, w4_ref, b4_ref,
             out_ref, acc_ref, acch_ref):
        j = pl.program_id(0)

        @pl.when(j == 0)
        def _():
            acc_ref[...] = jnp.zeros((G, D), jnp.float32)
            acch_ref[...] = jnp.zeros((G, H), jnp.float32)

        h8 = jnp.maximum(
            p0_ref[0, :, :H] + p1_ref[0, :, :H] + rp_ref[...] + b_ref[...], 0.0)
        e = emb_ref[...]
        bv = bat_ref[0, 0, :]
        mask = (bv[None, :] == lax.broadcasted_iota(jnp.int32, (G, RB), 0)
                ).astype(jnp.float32)
        acc_ref[...] += jnp.dot(mask, e, preferred_element_type=jnp.float32)
        acch_ref[...] += jnp.dot(mask, h8, preferred_element_type=jnp.float32)

        @pl.when(j == NB - 1)
        def _():
            h = (jnp.dot(acc_ref[...], w1_ref[...],
                         preferred_element_type=jnp.float32)
                 + jnp.dot(acch_ref[...], w1_ref[8 * H:9 * H, :],
                           preferred_element_type=jnp.float32)
                 + b1_ref[...])
            h = jnp.where(h >= 0, h, 0.1 * h)
            h = jnp.maximum(jnp.dot(h, w2_ref[...],
                                    preferred_element_type=jnp.float32)
                            + b2_ref[...], 0.0)
            h = jnp.maximum(jnp.dot(h, w3_ref[...],
                                    preferred_element_type=jnp.float32)
                            + b3_ref[...], 0.0)
            out_ref[...] = jnp.dot(h, w4_ref[...],
                                   preferred_element_type=jnp.float32) + b4_ref[...]

    return pl.pallas_call(
        body,
        grid=(NB,),
        in_specs=[
            pl.BlockSpec((RB, D), lambda j: (j, 0)),
            pl.BlockSpec((1, RB, PW), lambda j: (0, j, 0)),
            pl.BlockSpec((1, RB, PW), lambda j: (1, j, 0)),
            pl.BlockSpec((RB, H), lambda j: (j, 0)),
            pl.BlockSpec((1, H), lambda j: (0, 0)),
            pl.BlockSpec((1, 1, RB), lambda j: (j, 0, 0)),
            pl.BlockSpec((D, H), lambda j: (0, 0)),
            pl.BlockSpec((1, H), lambda j: (0, 0)),
            pl.BlockSpec((H, H), lambda j: (0, 0)),
            pl.BlockSpec((1, H), lambda j: (0, 0)),
            pl.BlockSpec((H, 256), lambda j: (0, 0)),
            pl.BlockSpec((1, 256), lambda j: (0, 0)),
            pl.BlockSpec((256, H), lambda j: (0, 0)),
            pl.BlockSpec((1, H), lambda j: (0, 0)),
        ],
        out_specs=pl.BlockSpec((G, H), lambda j: (0, 0)),
        out_shape=jax.ShapeDtypeStruct((G, H), jnp.float32),
        scratch_shapes=[
            pltpu.VMEM((G, D), jnp.float32),
            pltpu.VMEM((G, H), jnp.float32),
        ],
    )(emb, parts, parts, r7, b7, batch_r, w1p, b1, w2, b2, w3, b3, w4, b4)


def kernel(x, edge_index, batch, learnable_skip, pre_w, pre_b,
           ll_w_0, ll_b_0, lr_w_0, ll_w_1, ll_b_1, lr_w_1,
           ll_w_2, ll_b_2, lr_w_2, ll_w_3, ll_b_3, lr_w_3,
           ll_w_4, ll_b_4, lr_w_4, ll_w_5, ll_b_5, lr_w_5,
           ll_w_6, ll_b_6, lr_w_6, ll_w_7, ll_b_7, lr_w_7,
           post_w1, post_b1, post_w2, post_b2,
           post_w3, post_b3, post_w4, post_b4):
    ll_ws = [ll_w_0, ll_w_1, ll_w_2, ll_w_3, ll_w_4, ll_w_5, ll_w_6, ll_w_7]
    ll_bs = [ll_b_0, ll_b_1, ll_b_2, ll_b_3, ll_b_4, ll_b_5, ll_b_6, ll_b_7]
    lr_ws = [lr_w_0, lr_w_1, lr_w_2, lr_w_3, lr_w_4, lr_w_5, lr_w_6, lr_w_7]

    skip = jax.nn.sigmoid(learnable_skip)
    sll, slr = [], []
    for i in range(L):
        sc = jnp.repeat(skip[i, :i + 1], H)[:, None]
        sll.append(jnp.pad(ll_ws[i] * sc, ((0, D - (i + 1) * H), (0, 0))))
        slr.append(jnp.pad(lr_ws[i] * sc, ((0, D - (i + 1) * H), (0, 0))))

    src_rows = jnp.pad(edge_index[0], (0, EPAD - E)).reshape(EROWS, EC)
    dst_rows = jnp.pad(edge_index[1], (0, EPAD - E),
                       constant_values=N).reshape(EROWS, EC)
    zero_rows = jnp.zeros((NPAD, PW), jnp.float32)
    batch_r = batch.reshape(NB, 1, RB)

    emb, p, r = _pre_call(x, pre_w, pre_b.reshape(1, H),
                          sll[0][:H], slr[0][:H])
    for i in range(1, L):
        parts = _sc_agg(p, src_rows, dst_rows, zero_rows)
        emb, p, r = _layer_call(i, emb, parts, r,
                                ll_bs[i - 1].reshape(1, H), sll[i], slr[i])
    parts = _sc_agg(p, src_rows, dst_rows, zero_rows)
    return _final_call(emb, parts, r, ll_bs[L - 1].reshape(1, H), batch_r,
                       jnp.pad(post_w1, ((0, D - 9 * H), (0, 0))),
                       post_b1.reshape(1, H), post_w2, post_b2.reshape(1, H),
                       post_w3, post_b3.reshape(1, 256),
                       post_w4, post_b4.reshape(1, H))


# speedup vs baseline: 1.2486x; 1.1380x over previous
import functools

import jax
import jax.numpy as jnp
from jax import lax
from jax.experimental import pallas as pl
from jax.experimental.pallas import tpu as pltpu
from jax.experimental.pallas import tpu_sc as plsc

N = 10000
E = 320000
IN_DIM = 128
H = 64
L = 8
G = 16

D = 640
NB = 10
RB = 1000

PW = 128
NPAD = 10240
NPT = NPAD // 16
EPAD = 327680
EC = 128
EROWS = EPAD // EC
EPT = EROWS // 32
NPASS = 2
PR = EPT // NPASS


def _sc_agg(p, src_rows, dst_rows, zero_rows):
    mesh = plsc.VectorSubcoreMesh(core_axis_name="c", subcore_axis_name="s")

    @functools.partial(
        pl.kernel,
        mesh=mesh,
        out_type=jax.ShapeDtypeStruct((2, NPAD, PW), jnp.float32),
        scratch_types=[
            pltpu.VMEM((PR, EC), jnp.int32),
            pltpu.VMEM((PR, EC), jnp.int32),
            pltpu.VMEM((EC, PW), jnp.float32),
            pltpu.VMEM((EC, PW), jnp.float32),
            pltpu.VMEM_SHARED((NPAD, PW), jnp.float32),
            pltpu.SemaphoreType.DMA,
            pltpu.SemaphoreType.DMA,
        ],
    )
    def body(p_hbm, src_hbm, dst_hbm, zer_hbm, out_hbm,
             sidx, didx, rows0, rows1, acc, sem0, sem1):
        c = lax.axis_index("c")
        s = lax.axis_index("s")
        wid = c * 16 + s
        zr = s * NPT
        pltpu.sync_copy(zer_hbm.at[pl.ds(zr, NPT)], acc.at[pl.ds(zr, NPT)])
        r0 = wid * EPT
        plsc.subcore_barrier()

        def run_pass(base):
            pltpu.sync_copy(src_hbm.at[pl.ds(base, PR)], sidx)
            pltpu.sync_copy(dst_hbm.at[pl.ds(base, PR)], didx)
            pltpu.async_copy(p_hbm.at[sidx.at[0]], rows0, sem0)

            def step(jj, carry):
                j0 = 2 * jj
                j1 = j0 + 1
                pltpu.async_copy(p_hbm.at[sidx.at[j1]], rows1, sem1)
                pltpu.make_async_copy(p_hbm.at[sidx.at[j0]], rows0, sem0).wait()
                pltpu.sync_copy(rows0, acc.at[didx.at[j0]], add=True)

                @pl.when(jj < PR // 2 - 1)
                def _():
                    pltpu.async_copy(p_hbm.at[sidx.at[j0 + 2]], rows0, sem0)

                pltpu.make_async_copy(p_hbm.at[sidx.at[j1]], rows1, sem1).wait()
                pltpu.sync_copy(rows1, acc.at[didx.at[j1]], add=True)
                return carry

            lax.fori_loop(0, PR // 2, step, 0)

        run_pass(r0)
        run_pass(r0 + PR)
        plsc.subcore_barrier()
        pltpu.sync_copy(acc.at[pl.ds(zr, NPT)], out_hbm.at[c, pl.ds(zr, NPT)])

    return body(p, src_rows, dst_rows, zero_rows)


def _pre_call(x, pre_w, pre_b2, lw0, rw0):
    def body(x_ref, w_ref, b_ref, lw_ref, rw_ref, emb_ref, p_ref, r_ref):
        h0 = jnp.dot(x_ref[...], w_ref[...],
                     preferred_element_type=jnp.float32,
                     precision=lax.Precision.HIGHEST) + b_ref[...]
        emb_ref[...] = jnp.concatenate(
            [h0, jnp.zeros((RB, D - H), jnp.float32)], axis=1)
        p_ref[...] = jnp.concatenate(
            [jnp.dot(h0, lw_ref[...], preferred_element_type=jnp.float32,
                     precision=lax.Precision.HIGHEST),
             jnp.zeros((RB, PW - H), jnp.float32)], axis=1)
        r_ref[...] = jnp.dot(h0, rw_ref[...], preferred_element_type=jnp.float32,
                     precision=lax.Precision.HIGHEST)

    return pl.pallas_call(
        body,
        grid=(NB,),
        in_specs=[
            pl.BlockSpec((RB, IN_DIM), lambda j: (j, 0)),
            pl.BlockSpec((IN_DIM, H), lambda j: (0, 0)),
            pl.BlockSpec((1, H), lambda j: (0, 0)),
            pl.BlockSpec((H, H), lambda j: (0, 0)),
            pl.BlockSpec((H, H), lambda j: (0, 0)),
        ],
        out_specs=[
            pl.BlockSpec((RB, D), lambda j: (j, 0)),
            pl.BlockSpec((RB, PW), lambda j: (j, 0)),
            pl.BlockSpec((RB, H), lambda j: (j, 0)),
        ],
        out_shape=[
            jax.ShapeDtypeStruct((N, D), jnp.float32),
            jax.ShapeDtypeStruct((N, PW), jnp.float32),
            jax.ShapeDtypeStruct((N, H), jnp.float32),
        ],
    )(x, pre_w, pre_b2, lw0, rw0)


def _layer_call(i, emb, parts, rprev, bprev, lw, rw):
    lo = i * H

    def body(emb_ref, p0_ref, p1_ref, rp_ref, b_ref, lw_ref, rw_ref,
             embu_ref, p_ref, r_ref):
        hnew = jnp.maximum(
            p0_ref[0, :, :H] + p1_ref[0, :, :H] + rp_ref[...] + b_ref[...], 0.0)
        e = emb_ref[...]
        embu_ref[...] = e + jnp.concatenate(
            [jnp.zeros((RB, lo), jnp.float32), hnew,
             jnp.zeros((RB, D - lo - H), jnp.float32)], axis=1)
        p_ref[...] = jnp.concatenate(
            [jnp.dot(e, lw_ref[...], preferred_element_type=jnp.float32,
                     precision=lax.Precision.HIGHEST)
             + jnp.dot(hnew, lw_ref[lo:lo + H, :],
                       preferred_element_type=jnp.float32,
                     precision=lax.Precision.HIGHEST),
             jnp.zeros((RB, PW - H), jnp.float32)], axis=1)
        r_ref[...] = (jnp.dot(e, rw_ref[...], preferred_element_type=jnp.float32,
                     precision=lax.Precision.HIGHEST)
                      + jnp.dot(hnew, rw_ref[lo:lo + H, :],
                                preferred_element_type=jnp.float32,
                     precision=lax.Precision.HIGHEST))

    return pl.pallas_call(
        body,
        grid=(NB,),
        in_specs=[
            pl.BlockSpec((RB, D), lambda j: (j, 0)),
            pl.BlockSpec((1, RB, PW), lambda j: (0, j, 0)),
            pl.BlockSpec((1, RB, PW), lambda j: (1, j, 0)),
            pl.BlockSpec((RB, H), lambda j: (j, 0)),
            pl.BlockSpec((1, H), lambda j: (0, 0)),
            pl.BlockSpec((D, H), lambda j: (0, 0)),
            pl.BlockSpec((D, H), lambda j: (0, 0)),
        ],
        out_specs=[
            pl.BlockSpec((RB, D), lambda j: (j, 0)),
            pl.BlockSpec((RB, PW), lambda j: (j, 0)),
            pl.BlockSpec((RB, H), lambda j: (j, 0)),
        ],
        out_shape=[
            jax.ShapeDtypeStruct((N, D), jnp.float32),
            jax.ShapeDtypeStruct((N, PW), jnp.float32),
            jax.ShapeDtypeStruct((N, H), jnp.float32),
        ],
        input_output_aliases={0: 0},
    )(emb, parts, parts, rprev, bprev, lw, rw)


def _final_call(emb, parts, r7, b7, batch_r, w1p, b1, w2, b2, w3, b3, w4, b4):
    def body(emb_ref, p0_ref, p1_ref, rp_ref, b_ref, bat_ref,
             w1_ref, b1_ref, w2_ref, b2_ref, w3_ref, b3_ref, w4_ref, b4_ref,
             out_ref, acc_ref, acch_ref):
        j = pl.program_id(0)

        @pl.when(j == 0)
        def _():
            acc_ref[...] = jnp.zeros((G, D), jnp.float32)
            acch_ref[...] = jnp.zeros((G, H), jnp.float32)

        h8 = jnp.maximum(
            p0_ref[0, :, :H] + p1_ref[0, :, :H] + rp_ref[...] + b_ref[...], 0.0)
        e = emb_ref[...]
        bv = bat_ref[0, 0, :]
        mask = (bv[None, :] == lax.broadcasted_iota(jnp.int32, (G, RB), 0)
                ).astype(jnp.float32)
        acc_ref[...] += jnp.dot(mask, e, preferred_element_type=jnp.float32,
                     precision=lax.Precision.HIGHEST)
        acch_ref[...] += jnp.dot(mask, h8, preferred_element_type=jnp.float32,
                     precision=lax.Precision.HIGHEST)

        @pl.when(j == NB - 1)
        def _():
            h = (jnp.dot(acc_ref[...], w1_ref[...],
                         preferred_element_type=jnp.float32,
                     precision=lax.Precision.HIGHEST)
                 + jnp.dot(acch_ref[...], w1_ref[8 * H:9 * H, :],
                           preferred_element_type=jnp.float32,
                     precision=lax.Precision.HIGHEST)
                 + b1_ref[...])
            h = jnp.where(h >= 0, h, 0.1 * h)
            h = jnp.maximum(jnp.dot(h, w2_ref[...],
                                    preferred_element_type=jnp.float32,
                     precision=lax.Precision.HIGHEST)
                            + b2_ref[...], 0.0)
            h = jnp.maximum(jnp.dot(h, w3_ref[...],
                                    preferred_element_type=jnp.float32,
                     precision=lax.Precision.HIGHEST)
                            + b3_ref[...], 0.0)
            out_ref[...] = jnp.dot(h, w4_ref[...],
                                   preferred_element_type=jnp.float32,
                     precision=lax.Precision.HIGHEST) + b4_ref[...]

    return pl.pallas_call(
        body,
        grid=(NB,),
        in_specs=[
            pl.BlockSpec((RB, D), lambda j: (j, 0)),
            pl.BlockSpec((1, RB, PW), lambda j: (0, j, 0)),
            pl.BlockSpec((1, RB, PW), lambda j: (1, j, 0)),
            pl.BlockSpec((RB, H), lambda j: (j, 0)),
            pl.BlockSpec((1, H), lambda j: (0, 0)),
            pl.BlockSpec((1, 1, RB), lambda j: (j, 0, 0)),
            pl.BlockSpec((D, H), lambda j: (0, 0)),
            pl.BlockSpec((1, H), lambda j: (0, 0)),
            pl.BlockSpec((H, H), lambda j: (0, 0)),
            pl.BlockSpec((1, H), lambda j: (0, 0)),
            pl.BlockSpec((H, 256), lambda j: (0, 0)),
            pl.BlockSpec((1, 256), lambda j: (0, 0)),
            pl.BlockSpec((256, H), lambda j: (0, 0)),
            pl.BlockSpec((1, H), lambda j: (0, 0)),
        ],
        out_specs=pl.BlockSpec((G, H), lambda j: (0, 0)),
        out_shape=jax.ShapeDtypeStruct((G, H), jnp.float32),
        scratch_shapes=[
            pltpu.VMEM((G, D), jnp.float32),
            pltpu.VMEM((G, H), jnp.float32),
        ],
    )(emb, parts, parts, r7, b7, batch_r, w1p, b1, w2, b2, w3, b3, w4, b4)


def kernel(x, edge_index, batch, learnable_skip, pre_w, pre_b,
           ll_w_0, ll_b_0, lr_w_0, ll_w_1, ll_b_1, lr_w_1,
           ll_w_2, ll_b_2, lr_w_2, ll_w_3, ll_b_3, lr_w_3,
           ll_w_4, ll_b_4, lr_w_4, ll_w_5, ll_b_5, lr_w_5,
           ll_w_6, ll_b_6, lr_w_6, ll_w_7, ll_b_7, lr_w_7,
           post_w1, post_b1, post_w2, post_b2,
           post_w3, post_b3, post_w4, post_b4):
    ll_ws = [ll_w_0, ll_w_1, ll_w_2, ll_w_3, ll_w_4, ll_w_5, ll_w_6, ll_w_7]
    ll_bs = [ll_b_0, ll_b_1, ll_b_2, ll_b_3, ll_b_4, ll_b_5, ll_b_6, ll_b_7]
    lr_ws = [lr_w_0, lr_w_1, lr_w_2, lr_w_3, lr_w_4, lr_w_5, lr_w_6, lr_w_7]

    skip = jax.nn.sigmoid(learnable_skip)
    sll, slr = [], []
    for i in range(L):
        sc = jnp.repeat(skip[i, :i + 1], H)[:, None]
        sll.append(jnp.pad(ll_ws[i] * sc, ((0, D - (i + 1) * H), (0, 0))))
        slr.append(jnp.pad(lr_ws[i] * sc, ((0, D - (i + 1) * H), (0, 0))))

    src_rows = jnp.pad(edge_index[0], (0, EPAD - E)).reshape(EROWS, EC)
    dst_rows = jnp.pad(edge_index[1], (0, EPAD - E),
                       constant_values=N).reshape(EROWS, EC)
    zero_rows = jnp.zeros((NPAD, PW), jnp.float32)
    batch_r = batch.reshape(NB, 1, RB)

    emb, p, r = _pre_call(x, pre_w, pre_b.reshape(1, H),
                          sll[0][:H], slr[0][:H])
    for i in range(1, L):
        parts = _sc_agg(p, src_rows, dst_rows, zero_rows)
        emb, p, r = _layer_call(i, emb, parts, r,
                                ll_bs[i - 1].reshape(1, H), sll[i], slr[i])
    parts = _sc_agg(p, src_rows, dst_rows, zero_rows)
    return _final_call(emb, parts, r, ll_bs[L - 1].reshape(1, H), batch_r,
                       jnp.pad(post_w1, ((0, D - 9 * H), (0, 0))),
                       post_b1.reshape(1, H), post_w2, post_b2.reshape(1, H),
                       post_w3, post_b3.reshape(1, 256),
                       post_w4, post_b4.reshape(1, H))


# speedup vs baseline: 1.3994x; 1.1207x over previous
import functools

import jax
import jax.numpy as jnp
from jax import lax
from jax.experimental import pallas as pl
from jax.experimental.pallas import tpu as pltpu
from jax.experimental.pallas import tpu_sc as plsc

N = 10000
E = 320000
IN_DIM = 128
H = 64
L = 8
G = 16

D = 640
NB = 10
RB = 1000

PW = 128
NPAD = 10240
NPT = NPAD // 16
EPAD = 327680
EC = 128
EROWS = EPAD // EC
EPT = EROWS // 32
NPASS = 2
PR = EPT // NPASS


def _sc_agg(p, src_rows, dst_rows, zero_rows):
    mesh = plsc.VectorSubcoreMesh(core_axis_name="c", subcore_axis_name="s")

    @functools.partial(
        pl.kernel,
        mesh=mesh,
        out_type=jax.ShapeDtypeStruct((2, NPAD, PW), jnp.float32),
        scratch_types=[
            pltpu.VMEM((PR, EC), jnp.int32),
            pltpu.VMEM((PR, EC), jnp.int32),
            pltpu.VMEM((EC, PW), jnp.float32),
            pltpu.VMEM((EC, PW), jnp.float32),
            pltpu.VMEM_SHARED((NPAD, PW), jnp.float32),
            pltpu.SemaphoreType.DMA,
            pltpu.SemaphoreType.DMA,
        ],
    )
    def body(p_hbm, src_hbm, dst_hbm, zer_hbm, out_hbm,
             sidx, didx, rows0, rows1, acc, sem0, sem1):
        c = lax.axis_index("c")
        s = lax.axis_index("s")
        wid = c * 16 + s
        zr = s * NPT
        pltpu.sync_copy(zer_hbm.at[pl.ds(zr, NPT)], acc.at[pl.ds(zr, NPT)])
        r0 = wid * EPT
        plsc.subcore_barrier()

        def run_pass(base):
            pltpu.sync_copy(src_hbm.at[pl.ds(base, PR)], sidx)
            pltpu.sync_copy(dst_hbm.at[pl.ds(base, PR)], didx)
            pltpu.async_copy(p_hbm.at[sidx.at[0]], rows0, sem0)

            def step(jj, carry):
                j0 = 2 * jj
                j1 = j0 + 1
                pltpu.async_copy(p_hbm.at[sidx.at[j1]], rows1, sem1)
                pltpu.make_async_copy(p_hbm.at[sidx.at[j0]], rows0, sem0).wait()
                pltpu.sync_copy(rows0, acc.at[didx.at[j0]], add=True)

                @pl.when(jj < PR // 2 - 1)
                def _():
                    pltpu.async_copy(p_hbm.at[sidx.at[j0 + 2]], rows0, sem0)

                pltpu.make_async_copy(p_hbm.at[sidx.at[j1]], rows1, sem1).wait()
                pltpu.sync_copy(rows1, acc.at[didx.at[j1]], add=True)
                return carry

            lax.fori_loop(0, PR // 2, step, 0)

        run_pass(r0)
        run_pass(r0 + PR)
        plsc.subcore_barrier()
        pltpu.sync_copy(acc.at[pl.ds(zr, NPT)], out_hbm.at[c, pl.ds(zr, NPT)])

    return body(p, src_rows, dst_rows, zero_rows)


def _pre_call(x, pre_w, pre_b2, s00, lw0, rw0):
    def body(x_ref, w_ref, b_ref, s_ref, lw_ref, rw_ref, emb_ref, p_ref, r_ref):
        h0 = jnp.dot(x_ref[...], w_ref[...],
                     preferred_element_type=jnp.float32) + b_ref[...]
        emb_ref[...] = jnp.concatenate(
            [h0, jnp.zeros((RB, D - H), jnp.float32)], axis=1)
        c0 = h0 * s_ref[...]
        p_ref[...] = jnp.concatenate(
            [jnp.dot(c0, lw_ref[...], preferred_element_type=jnp.float32,
                     precision=lax.Precision.HIGHEST),
             jnp.zeros((RB, PW - H), jnp.float32)], axis=1)
        r_ref[...] = jnp.dot(c0, rw_ref[...], preferred_element_type=jnp.float32)

    return pl.pallas_call(
        body,
        grid=(NB,),
        in_specs=[
            pl.BlockSpec((RB, IN_DIM), lambda j: (j, 0)),
            pl.BlockSpec((IN_DIM, H), lambda j: (0, 0)),
            pl.BlockSpec((1, H), lambda j: (0, 0)),
            pl.BlockSpec((1, H), lambda j: (0, 0)),
            pl.BlockSpec((H, H), lambda j: (0, 0)),
            pl.BlockSpec((H, H), lambda j: (0, 0)),
        ],
        out_specs=[
            pl.BlockSpec((RB, D), lambda j: (j, 0)),
            pl.BlockSpec((RB, PW), lambda j: (j, 0)),
            pl.BlockSpec((RB, H), lambda j: (j, 0)),
        ],
        out_shape=[
            jax.ShapeDtypeStruct((N, D), jnp.float32),
            jax.ShapeDtypeStruct((N, PW), jnp.float32),
            jax.ShapeDtypeStruct((N, H), jnp.float32),
        ],
    )(x, pre_w, pre_b2, s00, lw0, rw0)


def _layer_call(i, emb, parts, rprev, bprev, sv, lw, rw):
    lo = i * H

    def body(emb_ref, p0_ref, p1_ref, rp_ref, b_ref, s_ref, lw_ref, rw_ref,
             embu_ref, p_ref, r_ref):
        hnew = jnp.maximum(
            p0_ref[0, :, :H] + p1_ref[0, :, :H] + rp_ref[...] + b_ref[...], 0.0)
        eu = emb_ref[...] + jnp.concatenate(
            [jnp.zeros((RB, lo), jnp.float32), hnew,
             jnp.zeros((RB, D - lo - H), jnp.float32)], axis=1)
        embu_ref[...] = eu
        curr = eu * s_ref[...]
        p_ref[...] = jnp.concatenate(
            [jnp.dot(curr, lw_ref[...], preferred_element_type=jnp.float32,
                     precision=lax.Precision.HIGHEST),
             jnp.zeros((RB, PW - H), jnp.float32)], axis=1)
        r_ref[...] = jnp.dot(curr, rw_ref[...],
                             preferred_element_type=jnp.float32)

    return pl.pallas_call(
        body,
        grid=(NB,),
        in_specs=[
            pl.BlockSpec((RB, D), lambda j: (j, 0)),
            pl.BlockSpec((1, RB, PW), lambda j: (0, j, 0)),
            pl.BlockSpec((1, RB, PW), lambda j: (1, j, 0)),
            pl.BlockSpec((RB, H), lambda j: (j, 0)),
            pl.BlockSpec((1, H), lambda j: (0, 0)),
            pl.BlockSpec((1, D), lambda j: (0, 0)),
            pl.BlockSpec((D, H), lambda j: (0, 0)),
            pl.BlockSpec((D, H), lambda j: (0, 0)),
        ],
        out_specs=[
            pl.BlockSpec((RB, D), lambda j: (j, 0)),
            pl.BlockSpec((RB, PW), lambda j: (j, 0)),
            pl.BlockSpec((RB, H), lambda j: (j, 0)),
        ],
        out_shape=[
            jax.ShapeDtypeStruct((N, D), jnp.float32),
            jax.ShapeDtypeStruct((N, PW), jnp.float32),
            jax.ShapeDtypeStruct((N, H), jnp.float32),
        ],
        input_output_aliases={0: 0},
    )(emb, parts, parts, rprev, bprev, sv, lw, rw)


def _final_call(emb, parts, r7, b7, batch_r, w1p, b1, w2, b2, w3, b3, w4, b4):
    def body(emb_ref, p0_ref, p1_ref, rp_ref, b_ref, bat_ref,
             w1_ref, b1_ref, w2_ref, b2_ref, w3_ref, b3_ref, w4_ref, b4_ref,
             out_ref, acc_ref, acch_ref):
        j = pl.program_id(0)

        @pl.when(j == 0)
        def _():
            acc_ref[...] = jnp.zeros((G, D), jnp.float32)
            acch_ref[...] = jnp.zeros((G, H), jnp.float32)

        h8 = jnp.maximum(
            p0_ref[0, :, :H] + p1_ref[0, :, :H] + rp_ref[...] + b_ref[...], 0.0)
        e = emb_ref[...]
        bv = bat_ref[0, 0, :]
        mask = (bv[None, :] == lax.broadcasted_iota(jnp.int32, (G, RB), 0)
                ).astype(jnp.float32)
        acc_ref[...] += jnp.dot(mask, e, preferred_element_type=jnp.float32,
                                precision=lax.Precision.HIGHEST)
        acch_ref[...] += jnp.dot(mask, h8, preferred_element_type=jnp.float32,
                                 precision=lax.Precision.HIGHEST)

        @pl.when(j == NB - 1)
        def _():
            h = (jnp.dot(acc_ref[...], w1_ref[...],
                         preferred_element_type=jnp.float32)
                 + jnp.dot(acch_ref[...], w1_ref[8 * H:9 * H, :],
                           preferred_element_type=jnp.float32)
                 + b1_ref[...])
            h = jnp.where(h >= 0, h, 0.1 * h)
            h = jnp.maximum(jnp.dot(h, w2_ref[...],
                                    preferred_element_type=jnp.float32)
                            + b2_ref[...], 0.0)
            h = jnp.maximum(jnp.dot(h, w3_ref[...],
                                    preferred_element_type=jnp.float32)
                            + b3_ref[...], 0.0)
            out_ref[...] = jnp.dot(h, w4_ref[...],
                                   preferred_element_type=jnp.float32) + b4_ref[...]

    return pl.pallas_call(
        body,
        grid=(NB,),
        in_specs=[
            pl.BlockSpec((RB, D), lambda j: (j, 0)),
            pl.BlockSpec((1, RB, PW), lambda j: (0, j, 0)),
            pl.BlockSpec((1, RB, PW), lambda j: (1, j, 0)),
            pl.BlockSpec((RB, H), lambda j: (j, 0)),
            pl.BlockSpec((1, H), lambda j: (0, 0)),
            pl.BlockSpec((1, 1, RB), lambda j: (j, 0, 0)),
            pl.BlockSpec((D, H), lambda j: (0, 0)),
            pl.BlockSpec((1, H), lambda j: (0, 0)),
            pl.BlockSpec((H, H), lambda j: (0, 0)),
            pl.BlockSpec((1, H), lambda j: (0, 0)),
            pl.BlockSpec((H, 256), lambda j: (0, 0)),
            pl.BlockSpec((1, 256), lambda j: (0, 0)),
            pl.BlockSpec((256, H), lambda j: (0, 0)),
            pl.BlockSpec((1, H), lambda j: (0, 0)),
        ],
        out_specs=pl.BlockSpec((G, H), lambda j: (0, 0)),
        out_shape=jax.ShapeDtypeStruct((G, H), jnp.float32),
        scratch_shapes=[
            pltpu.VMEM((G, D), jnp.float32),
            pltpu.VMEM((G, H), jnp.float32),
        ],
    )(emb, parts, parts, r7, b7, batch_r, w1p, b1, w2, b2, w3, b3, w4, b4)


def kernel(x, edge_index, batch, learnable_skip, pre_w, pre_b,
           ll_w_0, ll_b_0, lr_w_0, ll_w_1, ll_b_1, lr_w_1,
           ll_w_2, ll_b_2, lr_w_2, ll_w_3, ll_b_3, lr_w_3,
           ll_w_4, ll_b_4, lr_w_4, ll_w_5, ll_b_5, lr_w_5,
           ll_w_6, ll_b_6, lr_w_6, ll_w_7, ll_b_7, lr_w_7,
           post_w1, post_b1, post_w2, post_b2,
           post_w3, post_b3, post_w4, post_b4):
    ll_ws = [ll_w_0, ll_w_1, ll_w_2, ll_w_3, ll_w_4, ll_w_5, ll_w_6, ll_w_7]
    ll_bs = [ll_b_0, ll_b_1, ll_b_2, ll_b_3, ll_b_4, ll_b_5, ll_b_6, ll_b_7]
    lr_ws = [lr_w_0, lr_w_1, lr_w_2, lr_w_3, lr_w_4, lr_w_5, lr_w_6, lr_w_7]

    skip = jax.nn.sigmoid(learnable_skip)
    sll, slr, svs = [], [], []
    for i in range(L):
        sll.append(jnp.pad(ll_ws[i], ((0, D - (i + 1) * H), (0, 0))))
        slr.append(jnp.pad(lr_ws[i], ((0, D - (i + 1) * H), (0, 0))))
        svs.append(jnp.pad(jnp.repeat(skip[i, :i + 1], H),
                           (0, D - (i + 1) * H)).reshape(1, D))

    src_rows = jnp.pad(edge_index[0], (0, EPAD - E)).reshape(EROWS, EC)
    dst_rows = jnp.pad(edge_index[1], (0, EPAD - E),
                       constant_values=N).reshape(EROWS, EC)
    zero_rows = jnp.zeros((NPAD, PW), jnp.float32)
    batch_r = batch.reshape(NB, 1, RB)

    emb, p, r = _pre_call(x, pre_w, pre_b.reshape(1, H),
                          svs[0][:, :H], sll[0][:H], slr[0][:H])
    for i in range(1, L):
        parts = _sc_agg(p, src_rows, dst_rows, zero_rows)
        emb, p, r = _layer_call(i, emb, parts, r,
                                ll_bs[i - 1].reshape(1, H), svs[i],
                                sll[i], slr[i])
    parts = _sc_agg(p, src_rows, dst_rows, zero_rows)
    return _final_call(emb, parts, r, ll_bs[L - 1].reshape(1, H), batch_r,
                       jnp.pad(post_w1, ((0, D - 9 * H), (0, 0))),
                       post_b1.reshape(1, H), post_w2, post_b2.reshape(1, H),
                       post_w3, post_b3.reshape(1, 256),
                       post_w4, post_b4.reshape(1, H))
